# Initial kernel scaffold; baseline (speedup 1.0000x reference)
import jax
import jax.numpy as jnp
from jax.experimental import pallas as pl


def kernel(input, code, codebook):
    raise NotImplementedError("write your pallas kernel here")


# speedup vs baseline: 58.3707x; 58.3707x over previous
import functools

import jax
import jax.numpy as jnp
from jax import lax
from jax.experimental import pallas as pl
from jax.experimental.pallas import tpu as pltpu
from jax.experimental.pallas import tpu_sc as plsc

V = 100000
C = 16
W = 32
D = 64
K = C * W

_ROWS_PER_BLOCK = 2000

_NC = 2
_NS = 16
_NW = _NC * _NS

_B = 4096 * 50
_BPW = _B // _NW
_CHUNK = 640
_NCHUNK = _BPW // _CHUNK


def _table_matmul_body(code_ref, w_ref, out_ref):
    out_ref[...] = jnp.dot(code_ref[...], w_ref[...],
                           preferred_element_type=jnp.float32)


def _build_table(code2d, w):
    grid = V // _ROWS_PER_BLOCK
    return pl.pallas_call(
        _table_matmul_body,
        grid=(grid,),
        in_specs=[
            pl.BlockSpec((_ROWS_PER_BLOCK, K), lambda i: (i, 0)),
            pl.BlockSpec((K, D), lambda i: (0, 0)),
        ],
        out_specs=pl.BlockSpec((_ROWS_PER_BLOCK, D), lambda i: (i, 0)),
        out_shape=jax.ShapeDtypeStruct((V, D), jnp.float32),
    )(code2d, w)


def _gather_body(table_hbm, idx_hbm, out_hbm, idx_v, rows_v, sem0, sem1):
    wid = lax.axis_index("s") * _NC + lax.axis_index("c")
    base = wid * _BPW
    pltpu.sync_copy(idx_hbm.at[wid], idx_v)
    sems = (sem0, sem1)
    copies = [None, None]
    copies[0] = pltpu.async_copy(table_hbm.at[idx_v.at[0]], rows_v.at[0],
                                 sems[0])
    for j in range(1, _NCHUNK):
        b = j % 2
        copies[b] = pltpu.async_copy(table_hbm.at[idx_v.at[j]], rows_v.at[b],
                                     sems[b])
        copies[(j - 1) % 2].wait()
        pltpu.sync_copy(rows_v.at[(j - 1) % 2],
                        out_hbm.at[pl.ds(base + (j - 1) * _CHUNK, _CHUNK)])
    copies[(_NCHUNK - 1) % 2].wait()
    pltpu.sync_copy(rows_v.at[(_NCHUNK - 1) % 2],
                    out_hbm.at[pl.ds(base + (_NCHUNK - 1) * _CHUNK, _CHUNK)])


def _gather(table, idx3d):
    mesh = plsc.VectorSubcoreMesh(core_axis_name="c", subcore_axis_name="s")
    return pl.kernel(
        _gather_body,
        out_type=jax.ShapeDtypeStruct((_B, D), jnp.float32),
        mesh=mesh,
        scratch_types=[
            pltpu.VMEM((_NCHUNK, _CHUNK), jnp.int32),
            pltpu.VMEM((2, _CHUNK, D), jnp.float32),
            pltpu.SemaphoreType.DMA,
            pltpu.SemaphoreType.DMA,
        ],
        compiler_params=pltpu.CompilerParams(use_tc_tiling_on_sc=False),
    )(table, idx3d)


@jax.jit
def kernel(input, code, codebook):
    code2d = code.reshape(V, K)
    w = codebook.reshape(K, D)
    table = _build_table(code2d, w)
    idx3d = input.reshape(_NW, _NCHUNK, _CHUNK).astype(jnp.int32)
    out = _gather(table, idx3d)
    return out.reshape(input.shape[0], input.shape[1], D)
